# baseline (speedup 1.0000x reference)
import functools

import jax
import jax.numpy as jnp
from jax import lax
from jax.experimental import pallas as pl
from jax.experimental.pallas import tpu as pltpu
from jax.experimental.pallas import tpu_sc as plsc

NTOKEN = 1000000
NINP = 64
NC = 2
NS = 16
NW = NC * NS
CH = 128
NBUF = 4
KAHEAD = NBUF - 1
TBLK = 128
NFULL = NTOKEN // TBLK
NTAIL = NTOKEN - NFULL * TBLK


def _pack_body(wt_hbm, tail_hbm, w4_hbm, f0, f1, t0, t1, tlb,
               fs0, fs1, ts0, ts1):
    fbufs, tbufs = (f0, f1), (t0, t1)
    fsems, tsems = (fs0, fs1), (ts0, ts1)
    wid = lax.axis_index("s") * NC + lax.axis_index("c")
    per = NFULL // NW
    ext = NFULL - per * NW
    start = wid * per + jnp.minimum(wid, ext)
    cnt = per + jnp.where(wid < ext, 1, 0)

    rows = [lax.iota(jnp.int32, 16) + d0 for d0 in range(0, NINP, 16)]

    def fire(c, b):
        pltpu.async_copy(
            wt_hbm.at[:, pl.ds((start + c) * TBLK, TBLK)], fbufs[b],
            fsems[b])

    def wait_in(c, b):
        pltpu.make_async_copy(
            wt_hbm.at[:, pl.ds((start + c) * TBLK, TBLK)], fbufs[b],
            fsems[b]).wait()

    def put(c, b):
        pltpu.async_copy(
            tbufs[b], w4_hbm.at[pl.ds((start + c) * TBLK, TBLK)], tsems[b])

    def wait_put(c, b):
        pltpu.make_async_copy(
            tbufs[b], w4_hbm.at[pl.ds((start + c) * TBLK, TBLK)],
            tsems[b]).wait()

    def transpose(b):
        fb, tb = fbufs[b], tbufs[b]

        def tok(jj, carry):
            cols = jnp.zeros((16,), jnp.int32) + jj
            for i in range(NINP // 16):
                v = plsc.load_gather(fb, [rows[i], cols])
                tb[jj, pl.ds(16 * i, 16)] = v
            return carry
        lax.fori_loop(0, TBLK, tok, None)

    for b in range(2):
        pl.when(b < cnt)(functools.partial(fire, b, b))

    def step(c, carry):
        b0 = lax.rem(c, 2)

        def do(b):
            wait_in(c, b)
            pl.when(c >= 2)(functools.partial(wait_put, c - 2, b))
            transpose(b)
            put(c, b)
            pl.when(c + 2 < cnt)(functools.partial(fire, c + 2, b))
        pl.when(b0 == 0)(functools.partial(do, 0))
        pl.when(b0 == 1)(functools.partial(do, 1))
        return carry

    lax.fori_loop(0, cnt, step, None)
    for b in range(2):
        pl.when((cnt >= 2) & (lax.rem(cnt - 2, 2) == b))(
            functools.partial(wait_put, cnt - 2, b))
        pl.when((cnt >= 1) & (lax.rem(cnt - 1, 2) == b))(
            functools.partial(wait_put, cnt - 1, b))

    @pl.when(wid == NW - 1)
    def _():
        pltpu.sync_copy(tail_hbm, tlb)

        def tok(jj, carry):
            for i in range(NINP // 16):
                t0[jj, pl.ds(16 * i, 16)] = tlb[jj, pl.ds(16 * i, 16)]
            return carry
        lax.fori_loop(0, NTAIL, tok, None)
        pltpu.sync_copy(
            t0.at[pl.ds(0, NTAIL)], w4_hbm.at[pl.ds(NFULL * TBLK, NTAIL)])


def _embed_body(w4_hbm, p_hbm, out_hbm, p_v, *rest):
    n = p_hbm.shape[0] // NW
    nch = n // CH
    bufs = rest[:NBUF]
    cbufs = rest[NBUF:NBUF + 2]
    gsems = rest[NBUF + 2:2 * NBUF + 2]
    wsems = rest[2 * NBUF + 2:]

    wid = lax.axis_index("s") * NC + lax.axis_index("c")
    base = wid * n
    pltpu.sync_copy(p_hbm.at[pl.ds(base, n)], p_v)

    def fire(j, b):
        pltpu.async_copy(
            w4_hbm.at[p_v.at[pl.ds(j * CH, CH)]], bufs[b], gsems[b])

    def wait_gather(j, b):
        pltpu.make_async_copy(
            w4_hbm.at[p_v.at[pl.ds(j * CH, CH)]], bufs[b], gsems[b]).wait()

    def compact(b, cb):
        def row(r, carry):
            for i in range(NINP // 16):
                cbufs[cb][r, pl.ds(16 * i, 16)] = (
                    bufs[b][r, pl.ds(16 * i, 16)])
            return carry
        lax.fori_loop(0, CH, row, None)

    def put(j, cb):
        pltpu.async_copy(
            cbufs[cb], out_hbm.at[pl.ds((wid * nch + j) * CH, CH)],
            wsems[cb])

    def wait_put(j, cb):
        pltpu.make_async_copy(
            cbufs[cb], out_hbm.at[pl.ds((wid * nch + j) * CH, CH)],
            wsems[cb]).wait()

    for c in range(KAHEAD):
        fire(c, c)

    def group(g, carry):
        for bb in range(NBUF):
            j = g * NBUF + bb
            cb = bb % 2
            wait_gather(j, bb)
            pl.when(j >= 2)(lambda: wait_put(j - 2, cb))
            compact(bb, cb)
            put(j, cb)
            sl = (bb + KAHEAD) % NBUF
            pl.when(j + KAHEAD < nch)(lambda: fire(j + KAHEAD, sl))
        return carry

    lax.fori_loop(0, nch // NBUF, group, None)

    wait_put(nch - 2, (nch - 2) % 2)
    wait_put(nch - 1, (nch - 1) % 2)


_MESH = dict(core_axis_name="c", subcore_axis_name="s",
             num_cores=NC, num_subcores=NS)


def _make_pack():
    scratch = [pltpu.VMEM((NINP, TBLK), jnp.float32) for _ in range(2)]
    scratch += [pltpu.VMEM((TBLK, 128), jnp.float32) for _ in range(2)]
    scratch += [pltpu.VMEM((NTAIL, 128), jnp.float32)]
    scratch += [pltpu.SemaphoreType.DMA for _ in range(4)]
    return pl.kernel(
        _pack_body,
        out_type=jax.ShapeDtypeStruct((NTOKEN, 128), jnp.float32),
        mesh=plsc.VectorSubcoreMesh(**_MESH),
        scratch_types=scratch,
        compiler_params=pltpu.CompilerParams(
            use_tc_tiling_on_sc=True, needs_layout_passes=False),
    )


def _make_gather(n_total):
    per_w = n_total // NW
    scratch = [pltpu.VMEM((per_w,), jnp.int32)]
    scratch += [pltpu.VMEM((CH, 128), jnp.float32) for _ in range(NBUF)]
    scratch += [pltpu.VMEM((CH, NINP), jnp.float32) for _ in range(2)]
    scratch += [pltpu.SemaphoreType.DMA for _ in range(NBUF)]
    scratch += [pltpu.SemaphoreType.DMA for _ in range(2)]
    return pl.kernel(
        _embed_body,
        out_type=jax.ShapeDtypeStruct((n_total, NINP), jnp.float32),
        mesh=plsc.VectorSubcoreMesh(**_MESH),
        scratch_types=scratch,
        compiler_params=pltpu.CompilerParams(use_tc_tiling_on_sc=True),
    )


@jax.jit
def kernel(input, weight):
    b, l = input.shape
    n_total = b * l
    flat = input.reshape(n_total).astype(jnp.int32)
    tail = jnp.pad(weight[NFULL * TBLK:], ((0, 0), (0, 128 - NINP)))
    w4 = _make_pack()(weight.T, tail)
    out = _make_gather(n_total)(w4, flat)
    return out.reshape(b, l, NINP)


# speedup vs baseline: 1.9782x; 1.9782x over previous
import functools

import jax
import jax.numpy as jnp
from jax import lax
from jax.experimental import pallas as pl
from jax.experimental.pallas import tpu as pltpu
from jax.experimental.pallas import tpu_sc as plsc

NTOKEN = 1000000
NINP = 64
NC = 2
NS = 16
NW = NC * NS
CH = 128
NBUF = 4
KAHEAD = NBUF - 1
TBLK = 128
NFULL = NTOKEN // TBLK
NTAIL = NTOKEN - NFULL * TBLK


def _pack_body(wt_hbm, tail_hbm, w4_hbm, f0, f1, t0, t1, tlb,
               fs0, fs1, ts0, ts1):
    fbufs, tbufs = (f0, f1), (t0, t1)
    fsems, tsems = (fs0, fs1), (ts0, ts1)
    wid = lax.axis_index("s") * NC + lax.axis_index("c")
    per = NFULL // NW
    ext = NFULL - per * NW
    start = wid * per + jnp.minimum(wid, ext)
    cnt = per + jnp.where(wid < ext, 1, 0)

    rows = [lax.iota(jnp.int32, 16) + d0 for d0 in range(0, NINP, 16)]
    diag = [
        lax.rem(lax.iota(jnp.int32, 16) + s, jnp.full((16,), 16, jnp.int32))
        for s in range(16)
    ]

    def fire(c, b):
        pltpu.async_copy(
            wt_hbm.at[:, pl.ds((start + c) * TBLK, TBLK)], fbufs[b],
            fsems[b])

    def wait_in(c, b):
        pltpu.make_async_copy(
            wt_hbm.at[:, pl.ds((start + c) * TBLK, TBLK)], fbufs[b],
            fsems[b]).wait()

    def put(c, b):
        pltpu.async_copy(
            tbufs[b], w4_hbm.at[pl.ds((start + c) * TBLK, TBLK)], tsems[b])

    def wait_put(c, b):
        pltpu.make_async_copy(
            tbufs[b], w4_hbm.at[pl.ds((start + c) * TBLK, TBLK)],
            tsems[b]).wait()

    def transpose(b):
        fb, tb = fbufs[b], tbufs[b]

        def grp(g, carry):
            j0 = 16 * g
            for s in range(16):
                cols = j0 + diag[s]
                for i in range(NINP // 16):
                    v = plsc.load_gather(fb, [rows[i], cols])
                    plsc.store_scatter(tb, [cols, rows[i]], v)
            return carry
        lax.fori_loop(0, TBLK // 16, grp, None)

    for b in range(2):
        pl.when(b < cnt)(functools.partial(fire, b, b))

    def step(c, carry):
        b0 = lax.rem(c, 2)

        def do(b):
            wait_in(c, b)
            pl.when(c >= 2)(functools.partial(wait_put, c - 2, b))
            transpose(b)
            put(c, b)
            pl.when(c + 2 < cnt)(functools.partial(fire, c + 2, b))
        pl.when(b0 == 0)(functools.partial(do, 0))
        pl.when(b0 == 1)(functools.partial(do, 1))
        return carry

    lax.fori_loop(0, cnt, step, None)
    for b in range(2):
        pl.when((cnt >= 2) & (lax.rem(cnt - 2, 2) == b))(
            functools.partial(wait_put, cnt - 2, b))
        pl.when((cnt >= 1) & (lax.rem(cnt - 1, 2) == b))(
            functools.partial(wait_put, cnt - 1, b))

    @pl.when(wid == NW - 1)
    def _():
        pltpu.sync_copy(tail_hbm, tlb)

        def tok(jj, carry):
            for i in range(NINP // 16):
                t0[jj, pl.ds(16 * i, 16)] = tlb[jj, pl.ds(16 * i, 16)]
            return carry
        lax.fori_loop(0, NTAIL, tok, None)
        pltpu.sync_copy(
            t0.at[pl.ds(0, NTAIL)], w4_hbm.at[pl.ds(NFULL * TBLK, NTAIL)])


def _embed_body(w4_hbm, p_hbm, out_hbm, p_v, *rest):
    n = p_hbm.shape[0] // NW
    nch = n // CH
    bufs = rest[:NBUF]
    cbufs = rest[NBUF:NBUF + 2]
    gsems = rest[NBUF + 2:2 * NBUF + 2]
    wsems = rest[2 * NBUF + 2:]

    wid = lax.axis_index("s") * NC + lax.axis_index("c")
    base = wid * n
    pltpu.sync_copy(p_hbm.at[pl.ds(base, n)], p_v)

    def fire(j, b):
        pltpu.async_copy(
            w4_hbm.at[p_v.at[pl.ds(j * CH, CH)]], bufs[b], gsems[b])

    def wait_gather(j, b):
        pltpu.make_async_copy(
            w4_hbm.at[p_v.at[pl.ds(j * CH, CH)]], bufs[b], gsems[b]).wait()

    def compact(b, cb):
        def row(r, carry):
            for i in range(NINP // 16):
                cbufs[cb][r, pl.ds(16 * i, 16)] = (
                    bufs[b][r, pl.ds(16 * i, 16)])
            return carry
        lax.fori_loop(0, CH, row, None)

    def put(j, cb):
        pltpu.async_copy(
            cbufs[cb], out_hbm.at[pl.ds((wid * nch + j) * CH, CH)],
            wsems[cb])

    def wait_put(j, cb):
        pltpu.make_async_copy(
            cbufs[cb], out_hbm.at[pl.ds((wid * nch + j) * CH, CH)],
            wsems[cb]).wait()

    for c in range(KAHEAD):
        fire(c, c)

    def group(g, carry):
        for bb in range(NBUF):
            j = g * NBUF + bb
            cb = bb % 2
            wait_gather(j, bb)
            pl.when(j >= 2)(lambda: wait_put(j - 2, cb))
            compact(bb, cb)
            put(j, cb)
            sl = (bb + KAHEAD) % NBUF
            pl.when(j + KAHEAD < nch)(lambda: fire(j + KAHEAD, sl))
        return carry

    lax.fori_loop(0, nch // NBUF, group, None)

    wait_put(nch - 2, (nch - 2) % 2)
    wait_put(nch - 1, (nch - 1) % 2)


_MESH = dict(core_axis_name="c", subcore_axis_name="s",
             num_cores=NC, num_subcores=NS)


def _make_pack():
    scratch = [pltpu.VMEM((NINP, TBLK), jnp.float32) for _ in range(2)]
    scratch += [pltpu.VMEM((TBLK, 128), jnp.float32) for _ in range(2)]
    scratch += [pltpu.VMEM((NTAIL, 128), jnp.float32)]
    scratch += [pltpu.SemaphoreType.DMA for _ in range(4)]
    return pl.kernel(
        _pack_body,
        out_type=jax.ShapeDtypeStruct((NTOKEN, 128), jnp.float32),
        mesh=plsc.VectorSubcoreMesh(**_MESH),
        scratch_types=scratch,
        compiler_params=pltpu.CompilerParams(
            use_tc_tiling_on_sc=True, needs_layout_passes=False),
    )


def _make_gather(n_total):
    per_w = n_total // NW
    scratch = [pltpu.VMEM((per_w,), jnp.int32)]
    scratch += [pltpu.VMEM((CH, 128), jnp.float32) for _ in range(NBUF)]
    scratch += [pltpu.VMEM((CH, NINP), jnp.float32) for _ in range(2)]
    scratch += [pltpu.SemaphoreType.DMA for _ in range(NBUF)]
    scratch += [pltpu.SemaphoreType.DMA for _ in range(2)]
    return pl.kernel(
        _embed_body,
        out_type=jax.ShapeDtypeStruct((n_total, NINP), jnp.float32),
        mesh=plsc.VectorSubcoreMesh(**_MESH),
        scratch_types=scratch,
        compiler_params=pltpu.CompilerParams(use_tc_tiling_on_sc=True),
    )


@jax.jit
def kernel(input, weight):
    b, l = input.shape
    n_total = b * l
    flat = input.reshape(n_total).astype(jnp.int32)
    tail = jnp.pad(weight[NFULL * TBLK:], ((0, 0), (0, 128 - NINP)))
    w4 = _make_pack()(weight.T, tail)
    out = _make_gather(n_total)(w4, flat)
    return out.reshape(b, l, NINP)


# speedup vs baseline: 2.0087x; 1.0154x over previous
import functools

import jax
import jax.numpy as jnp
from jax import lax
from jax.experimental import pallas as pl
from jax.experimental.pallas import tpu as pltpu
from jax.experimental.pallas import tpu_sc as plsc

NTOKEN = 1000000
NINP = 64
NC = 2
NS = 16
NW = NC * NS
CH = 128
NBUF = 4
KAHEAD = NBUF - 1
TBLK = 256
NFULL = NTOKEN // TBLK
NTAIL = NTOKEN - NFULL * TBLK


def _pack_body(wt_hbm, tail_hbm, w4_hbm, f0, f1, t0, t1, tlb,
               fs0, fs1, ts0, ts1):
    fbufs, tbufs = (f0, f1), (t0, t1)
    fsems, tsems = (fs0, fs1), (ts0, ts1)
    wid = lax.axis_index("s") * NC + lax.axis_index("c")
    per = NFULL // NW
    ext = NFULL - per * NW
    start = wid * per + jnp.minimum(wid, ext)
    cnt = per + jnp.where(wid < ext, 1, 0)

    rows = [lax.iota(jnp.int32, 16) + d0 for d0 in range(0, NINP, 16)]
    diag = [
        lax.rem(lax.iota(jnp.int32, 16) + s, jnp.full((16,), 16, jnp.int32))
        for s in range(16)
    ]

    def fire(c, b):
        pltpu.async_copy(
            wt_hbm.at[:, pl.ds((start + c) * TBLK, TBLK)], fbufs[b],
            fsems[b])

    def wait_in(c, b):
        pltpu.make_async_copy(
            wt_hbm.at[:, pl.ds((start + c) * TBLK, TBLK)], fbufs[b],
            fsems[b]).wait()

    def put(c, b):
        pltpu.async_copy(
            tbufs[b], w4_hbm.at[pl.ds((start + c) * TBLK, TBLK)], tsems[b])

    def wait_put(c, b):
        pltpu.make_async_copy(
            tbufs[b], w4_hbm.at[pl.ds((start + c) * TBLK, TBLK)],
            tsems[b]).wait()

    def transpose(b):
        fb, tb = fbufs[b], tbufs[b]

        def grp(g, carry):
            j0 = 16 * g
            for s in range(16):
                cols = j0 + diag[s]
                for i in range(NINP // 16):
                    v = plsc.load_gather(fb, [rows[i], cols])
                    plsc.store_scatter(tb, [cols, rows[i]], v)
            return carry
        lax.fori_loop(0, TBLK // 16, grp, None)

    for b in range(2):
        pl.when(b < cnt)(functools.partial(fire, b, b))

    def step(c, carry):
        b0 = lax.rem(c, 2)

        def do(b):
            wait_in(c, b)
            pl.when(c >= 2)(functools.partial(wait_put, c - 2, b))
            transpose(b)
            put(c, b)
            pl.when(c + 2 < cnt)(functools.partial(fire, c + 2, b))
        pl.when(b0 == 0)(functools.partial(do, 0))
        pl.when(b0 == 1)(functools.partial(do, 1))
        return carry

    lax.fori_loop(0, cnt, step, None)
    for b in range(2):
        pl.when((cnt >= 2) & (lax.rem(cnt - 2, 2) == b))(
            functools.partial(wait_put, cnt - 2, b))
        pl.when((cnt >= 1) & (lax.rem(cnt - 1, 2) == b))(
            functools.partial(wait_put, cnt - 1, b))

    @pl.when(wid == NW - 1)
    def _():
        pltpu.sync_copy(tail_hbm, tlb)

        def tok(jj, carry):
            for i in range(NINP // 16):
                t0[jj, pl.ds(16 * i, 16)] = tlb[jj, pl.ds(16 * i, 16)]
            return carry
        lax.fori_loop(0, NTAIL, tok, None)
        pltpu.sync_copy(
            t0.at[pl.ds(0, NTAIL)], w4_hbm.at[pl.ds(NFULL * TBLK, NTAIL)])


def _embed_body(w4_hbm, p_hbm, out_hbm, p_v, *rest):
    n = p_hbm.shape[0] // NW
    nch = n // CH
    bufs = rest[:NBUF]
    cbufs = rest[NBUF:NBUF + 2]
    gsems = rest[NBUF + 2:2 * NBUF + 2]
    wsems = rest[2 * NBUF + 2:]

    wid = lax.axis_index("s") * NC + lax.axis_index("c")
    base = wid * n
    pltpu.sync_copy(p_hbm.at[pl.ds(base, n)], p_v)

    def fire(j, b):
        pltpu.async_copy(
            w4_hbm.at[p_v.at[pl.ds(j * CH, CH)]], bufs[b], gsems[b])

    def wait_gather(j, b):
        pltpu.make_async_copy(
            w4_hbm.at[p_v.at[pl.ds(j * CH, CH)]], bufs[b], gsems[b]).wait()

    def compact(b, cb):
        def row(r, carry):
            for i in range(NINP // 16):
                cbufs[cb][r, pl.ds(16 * i, 16)] = (
                    bufs[b][r, pl.ds(16 * i, 16)])
            return carry
        lax.fori_loop(0, CH, row, None)

    def put(j, cb):
        pltpu.async_copy(
            cbufs[cb], out_hbm.at[pl.ds((wid * nch + j) * CH, CH)],
            wsems[cb])

    def wait_put(j, cb):
        pltpu.make_async_copy(
            cbufs[cb], out_hbm.at[pl.ds((wid * nch + j) * CH, CH)],
            wsems[cb]).wait()

    for c in range(KAHEAD):
        fire(c, c)

    def group(g, carry):
        for bb in range(NBUF):
            j = g * NBUF + bb
            cb = bb % 2
            wait_gather(j, bb)
            pl.when(j >= 2)(lambda: wait_put(j - 2, cb))
            compact(bb, cb)
            put(j, cb)
            sl = (bb + KAHEAD) % NBUF
            pl.when(j + KAHEAD < nch)(lambda: fire(j + KAHEAD, sl))
        return carry

    lax.fori_loop(0, nch // NBUF, group, None)

    wait_put(nch - 2, (nch - 2) % 2)
    wait_put(nch - 1, (nch - 1) % 2)


_MESH = dict(core_axis_name="c", subcore_axis_name="s",
             num_cores=NC, num_subcores=NS)


def _make_pack():
    scratch = [pltpu.VMEM((NINP, TBLK), jnp.float32) for _ in range(2)]
    scratch += [pltpu.VMEM((TBLK, 128), jnp.float32) for _ in range(2)]
    scratch += [pltpu.VMEM((NTAIL, 128), jnp.float32)]
    scratch += [pltpu.SemaphoreType.DMA for _ in range(4)]
    return pl.kernel(
        _pack_body,
        out_type=jax.ShapeDtypeStruct((NTOKEN, 128), jnp.float32),
        mesh=plsc.VectorSubcoreMesh(**_MESH),
        scratch_types=scratch,
        compiler_params=pltpu.CompilerParams(
            use_tc_tiling_on_sc=True, needs_layout_passes=False),
    )


def _make_gather(n_total):
    per_w = n_total // NW
    scratch = [pltpu.VMEM((per_w,), jnp.int32)]
    scratch += [pltpu.VMEM((CH, 128), jnp.float32) for _ in range(NBUF)]
    scratch += [pltpu.VMEM((CH, NINP), jnp.float32) for _ in range(2)]
    scratch += [pltpu.SemaphoreType.DMA for _ in range(NBUF)]
    scratch += [pltpu.SemaphoreType.DMA for _ in range(2)]
    return pl.kernel(
        _embed_body,
        out_type=jax.ShapeDtypeStruct((n_total, NINP), jnp.float32),
        mesh=plsc.VectorSubcoreMesh(**_MESH),
        scratch_types=scratch,
        compiler_params=pltpu.CompilerParams(use_tc_tiling_on_sc=True),
    )


@jax.jit
def kernel(input, weight):
    b, l = input.shape
    n_total = b * l
    flat = input.reshape(n_total).astype(jnp.int32)
    tail = jnp.pad(weight[NFULL * TBLK:], ((0, 0), (0, 128 - NINP)))
    w4 = _make_pack()(weight.T, tail)
    out = _make_gather(n_total)(w4, flat)
    return out.reshape(b, l, NINP)
